# baseline (speedup 1.0000x reference)
import jax
import jax.numpy as jnp
from jax.experimental import pallas as pl
from jax.experimental.pallas import tpu as pltpu

_SLOTS = 8
_PREFETCH = 6


def _drqn_body(x_hbm, h0_ref, wih0_hbm, wih_hbm, whh_hbm, bih_ref, bhh_ref,
               wq_hbm, bq_ref,
               out_hbm, hout_ref, ring_ref, w_ref, wq_ref, qstage_ref,
               wih0_ref, wih_ref, whh_ref, wq_raw_ref,
               sems, sem_out, sem_w):
    B, T, D = x_hbm.shape
    H = h0_ref.shape[2]
    R = wq_raw_ref.shape[0]

    w_moves = [(wih0_hbm, wih0_ref), (whh_hbm, whh_ref),
               (wih_hbm, wih_ref), (wq_hbm, wq_raw_ref)]
    for i, (src_w, dst_w) in enumerate(w_moves):
        pltpu.make_async_copy(src_w, dst_w, sem_w.at[i]).start()
    f32 = jnp.float32
    bf16 = jnp.bfloat16
    out_tc = qstage_ref.shape[2]

    def out_chunk_refs(c):
        size = min(out_tc, T - c * out_tc)
        src_q = qstage_ref.at[c % 2, :, pl.ds(0, size)]
        dst_q = out_hbm.at[:, pl.ds(c * out_tc, size)]
        return src_q, dst_q

    flushed = []

    def flush_out(c):
        src_q, dst_q = out_chunk_refs(c)
        pltpu.make_async_copy(src_q, dst_q, sem_out.at[c % 2]).start()
        flushed.append(c)

    def wait_out(c):
        src_q, dst_q = out_chunk_refs(c)
        pltpu.make_async_copy(src_q, dst_q, sem_out.at[c % 2]).wait()

    def write_q(s, q):
        c, pos = s // out_tc, s % out_tc
        if pos == 0 and c >= 2:
            wait_out(c - 2)
        qstage_ref[c % 2, :, pos, :] = q
        if s == T - 1 or pos == out_tc - 1:
            flush_out(c)

    def start_fetch(t):
        pltpu.make_async_copy(
            x_hbm.at[:, t], ring_ref.at[t % _SLOTS],
            sems.at[t % _SLOTS]).start()

    def wait_fetch(t):
        slot = t % _SLOTS
        pltpu.make_async_copy(
            ring_ref.at[slot], ring_ref.at[slot], sems.at[slot]).wait()

    for t in range(min(_PREFETCH, T)):
        start_fetch(t)

    for i, (src_w, dst_w) in enumerate(w_moves):
        pltpu.make_async_copy(dst_w, dst_w, sem_w.at[i]).wait()
        if i == 0:
            w_ref[:D, :H] = wih0_ref[...].T.astype(bf16)
        elif i == 1:
            w_ref[D:, :H] = whh_ref[0].T.astype(bf16)
            w_ref[D:, H:] = whh_ref[1].T.astype(bf16)
        elif i == 2:
            w_ref[:D, H:] = wih_ref[1].T.astype(bf16)
        else:
            wq_ref[...] = wq_raw_ref[...].T.astype(bf16)

    w1 = w_ref[:, :H]
    w2 = w_ref[:, H:]
    wq = wq_ref[...]
    b1 = (bih_ref[0] + bhh_ref[0]).reshape(1, H)
    b2 = (bih_ref[1] + bhh_ref[1]).reshape(1, H)
    bq = bq_ref[...]

    h1b = h0_ref[0].astype(bf16)
    h2b = h0_ref[1].astype(bf16)

    for r in range(T + 1):
        h1b_old = h1b
        if r >= 2:
            q = jnp.dot(h2b, wq, preferred_element_type=f32) + bq
            write_q(r - 2, q)
        if r < T:
            wait_fetch(r)
            xb = ring_ref[r % _SLOTS].astype(bf16)
            xh = jnp.concatenate([xb, h1b_old], axis=1)
            h1 = jnp.tanh(
                jnp.dot(xh, w1, preferred_element_type=f32) + b1)
            h1b = h1.astype(bf16)
            if r + _PREFETCH < T:
                start_fetch(r + _PREFETCH)
            if r == T - 1:
                hout_ref[0] = h1
        if r > 0:
            hh = jnp.concatenate([h1b_old, h2b], axis=1)
            h2 = jnp.tanh(
                jnp.dot(hh, w2, preferred_element_type=f32) + b2)
            h2b = h2.astype(bf16)
            if r == T:
                hout_ref[1] = h2
    q = jnp.dot(h2b, wq, preferred_element_type=f32) + bq
    write_q(T - 1, q)
    for c in flushed[-2:]:
        wait_out(c)


def kernel(inputs, hidden_state, w_ih0, w_ih, w_hh, b_ih, b_hh, w_q, b_q):
    B, T, D = inputs.shape
    L, _, H = hidden_state.shape
    R = w_q.shape[0]

    full = lambda shape: pl.BlockSpec(shape, lambda: (0,) * len(shape))

    out, h_out = pl.pallas_call(
        _drqn_body,
        grid=(),
        in_specs=[
            pl.BlockSpec(memory_space=pl.ANY),
            full((L, B, H)),
            pl.BlockSpec(memory_space=pl.ANY),
            pl.BlockSpec(memory_space=pl.ANY),
            pl.BlockSpec(memory_space=pl.ANY),
            full((L, H)),
            full((L, H)),
            pl.BlockSpec(memory_space=pl.ANY),
            full((1, R)),
        ],
        out_specs=(
            pl.BlockSpec(memory_space=pl.ANY),
            full((L, B, H)),
        ),
        out_shape=(
            jax.ShapeDtypeStruct((B, T, R), jnp.float32),
            jax.ShapeDtypeStruct((L, B, H), jnp.float32),
        ),
        scratch_shapes=[
            pltpu.VMEM((_SLOTS, B, D), jnp.float32),
            pltpu.VMEM((D + H, 2 * H), jnp.bfloat16),
            pltpu.VMEM((H, R), jnp.bfloat16),
            pltpu.VMEM((2, B, min(16, T), R), jnp.float32),
            pltpu.VMEM((H, D), jnp.float32),
            pltpu.VMEM((L, H, H), jnp.float32),
            pltpu.VMEM((L, H, H), jnp.float32),
            pltpu.VMEM((R, H), jnp.float32),
            pltpu.SemaphoreType.DMA((_SLOTS,)),
            pltpu.SemaphoreType.DMA((2,)),
            pltpu.SemaphoreType.DMA((4,)),
        ],
    )(inputs, hidden_state, w_ih0, w_ih, w_hh, b_ih, b_hh, w_q,
      b_q.reshape(1, R))

    return out, h_out


# speedup vs baseline: 1.0132x; 1.0132x over previous
import jax
import jax.numpy as jnp
from jax.experimental import pallas as pl
from jax.experimental.pallas import tpu as pltpu

_SLOTS = 8
_PREFETCH = 6


def _drqn_body(x_hbm, h0_ref, wih0_ref, wih_ref, whh_ref, bih_ref, bhh_ref,
               wq_raw_ref, bq_ref,
               out_hbm, hout_ref, ring_ref, w_ref, wq_ref, qstage_ref,
               sems, sem_out):
    B, T, D = x_hbm.shape
    H = h0_ref.shape[2]
    R = wq_raw_ref.shape[0]
    f32 = jnp.float32
    bf16 = jnp.bfloat16
    out_tc = qstage_ref.shape[2]

    def out_chunk_refs(c):
        size = min(out_tc, T - c * out_tc)
        src_q = qstage_ref.at[c % 2, :, pl.ds(0, size)]
        dst_q = out_hbm.at[:, pl.ds(c * out_tc, size)]
        return src_q, dst_q

    flushed = []

    def flush_out(c):
        src_q, dst_q = out_chunk_refs(c)
        pltpu.make_async_copy(src_q, dst_q, sem_out.at[c % 2]).start()
        flushed.append(c)

    def wait_out(c):
        src_q, dst_q = out_chunk_refs(c)
        pltpu.make_async_copy(src_q, dst_q, sem_out.at[c % 2]).wait()

    def write_q(s, q):
        c, pos = s // out_tc, s % out_tc
        if pos == 0 and c >= 2:
            wait_out(c - 2)
        qstage_ref[c % 2, :, pos, :] = q
        if s == T - 1 or pos == out_tc - 1:
            flush_out(c)

    def start_fetch(t):
        pltpu.make_async_copy(
            x_hbm.at[:, t], ring_ref.at[t % _SLOTS],
            sems.at[t % _SLOTS]).start()

    def wait_fetch(t):
        slot = t % _SLOTS
        pltpu.make_async_copy(
            ring_ref.at[slot], ring_ref.at[slot], sems.at[slot]).wait()

    for t in range(min(_PREFETCH, T)):
        start_fetch(t)

    w_ref[:D, :H] = wih0_ref[...].T.astype(bf16)
    w_ref[D:, :H] = whh_ref[0].T.astype(bf16)
    w_ref[:D, H:] = wih_ref[1].T.astype(bf16)
    w_ref[D:, H:] = whh_ref[1].T.astype(bf16)
    wq_ref[...] = wq_raw_ref[...].T.astype(bf16)

    w1 = w_ref[:, :H]
    w2 = w_ref[:, H:]
    wq = wq_ref[...]
    b1 = (bih_ref[0] + bhh_ref[0]).reshape(1, H)
    b2 = (bih_ref[1] + bhh_ref[1]).reshape(1, H)
    bq = bq_ref[...]

    h1b = h0_ref[0].astype(bf16)
    h2b = h0_ref[1].astype(bf16)

    for r in range(T + 1):
        h1b_old = h1b
        if r >= 2:
            q = jnp.dot(h2b, wq, preferred_element_type=f32) + bq
            write_q(r - 2, q)
        if r < T:
            wait_fetch(r)
            xb = ring_ref[r % _SLOTS].astype(bf16)
            xh = jnp.concatenate([xb, h1b_old], axis=1)
            h1 = jnp.tanh(
                jnp.dot(xh, w1, preferred_element_type=f32) + b1)
            h1b = h1.astype(bf16)
            if r + _PREFETCH < T:
                start_fetch(r + _PREFETCH)
            if r == T - 1:
                hout_ref[0] = h1
        if r > 0:
            hh = jnp.concatenate([h1b_old, h2b], axis=1)
            h2 = jnp.tanh(
                jnp.dot(hh, w2, preferred_element_type=f32) + b2)
            h2b = h2.astype(bf16)
            if r == T:
                hout_ref[1] = h2
    q = jnp.dot(h2b, wq, preferred_element_type=f32) + bq
    write_q(T - 1, q)
    for c in flushed[-2:]:
        wait_out(c)


def kernel(inputs, hidden_state, w_ih0, w_ih, w_hh, b_ih, b_hh, w_q, b_q):
    B, T, D = inputs.shape
    L, _, H = hidden_state.shape
    R = w_q.shape[0]

    full = lambda shape: pl.BlockSpec(shape, lambda: (0,) * len(shape))

    out, h_out = pl.pallas_call(
        _drqn_body,
        grid=(),
        in_specs=[
            pl.BlockSpec(memory_space=pl.ANY),
            full((L, B, H)),
            full((H, D)),
            full((L, H, H)),
            full((L, H, H)),
            full((L, H)),
            full((L, H)),
            full((R, H)),
            full((1, R)),
        ],
        out_specs=(
            pl.BlockSpec(memory_space=pl.ANY),
            full((L, B, H)),
        ),
        out_shape=(
            jax.ShapeDtypeStruct((B, T, R), jnp.float32),
            jax.ShapeDtypeStruct((L, B, H), jnp.float32),
        ),
        scratch_shapes=[
            pltpu.VMEM((_SLOTS, B, D), jnp.float32),
            pltpu.VMEM((D + H, 2 * H), jnp.bfloat16),
            pltpu.VMEM((H, R), jnp.bfloat16),
            pltpu.VMEM((2, B, min(16, T), R), jnp.float32),
            pltpu.SemaphoreType.DMA((_SLOTS,)),
            pltpu.SemaphoreType.DMA((2,)),
        ],
    )(inputs, hidden_state, w_ih0, w_ih, w_hh, b_ih, b_hh, w_q,
      b_q.reshape(1, R))

    return out, h_out
